# baseline (speedup 1.0000x reference)
import functools

import jax
import jax.numpy as jnp
from jax import lax
from jax.experimental import pallas as pl
from jax.experimental.pallas import tpu as pltpu

_NCHUNK = 8


def _fused_body(xb_ref, sel_ref, w1_ref, w2_ref, g1_ref, b1_ref, g2_ref,
                b2_ref, o_ref, y1_ref, y2_ref, st_ref, *, eps):
    p = pl.program_id(0)
    j = pl.program_id(1)
    Nc, C, Hp, W4 = xb_ref.shape
    Wp = W4 // 4
    HW = Hp * Wp
    inv_count = 1.0 / float(_NCHUNK * Nc * HW)

    def conv3x3(a, w_ref):
        apad = jnp.pad(a, ((0, 0), (1, 1), (1, 1), (0, 0)))
        b = jnp.concatenate([apad[:, :, dx:dx + Wp, :] for dx in range(3)],
                            axis=3)
        z = jnp.dot(b.reshape(Nc * (Hp + 2) * Wp, 3 * C), w_ref[...],
                    preferred_element_type=jnp.float32)
        z = z.reshape(Nc, Hp + 2, Wp, 3 * C)
        return (z[:, 0:Hp, :, 0:C] + z[:, 1:Hp + 1, :, C:2 * C]
                + z[:, 2:Hp + 2, :, 2 * C:3 * C])

    def coeffs(srow, ssrow, g_ref, b_ref):
        mean = st_ref[srow:srow + 1, :] * inv_count
        var = jnp.maximum(st_ref[ssrow:ssrow + 1, :] * inv_count
                          - mean * mean, 0.0)
        scale = g_ref[...] * lax.rsqrt(var + eps)
        shift = b_ref[...] - mean * scale
        return scale.reshape(1, 1, 1, C), shift.reshape(1, 1, 1, C)

    @pl.when(p == 0)
    def _phase0():
        @pl.when(j == 0)
        def _init():
            st_ref[...] = jnp.zeros_like(st_ref)

        xv = xb_ref[...]
        vert = jnp.maximum(xv[..., 0:2 * Wp], xv[..., 2 * Wp:4 * Wp])
        vd = vert.astype(jnp.bfloat16).reshape(Nc * C * Hp, 2 * Wp)
        pc = jnp.dot(vd, sel_ref[...],
                     preferred_element_type=jnp.float32)
        pooled = jnp.maximum(pc[:, 0:Wp], pc[:, Wp:2 * Wp])
        xp = jnp.transpose(pooled.astype(jnp.bfloat16)
                           .reshape(Nc, C, Hp, Wp), (0, 2, 3, 1))
        y1 = conv3x3(xp, w1_ref)
        y1_ref[j] = y1
        st_ref[0:1, :] += jnp.sum(y1, axis=(0, 1, 2)).reshape(1, C)
        st_ref[1:2, :] += jnp.sum(y1 * y1, axis=(0, 1, 2)).reshape(1, C)

    @pl.when(p == 1)
    def _phase1():
        sc, sh = coeffs(0, 1, g1_ref, b1_ref)
        a1 = jnp.maximum(y1_ref[j] * sc + sh, 0.0).astype(jnp.bfloat16)
        y2 = conv3x3(a1, w2_ref)
        y2_ref[j] = y2
        st_ref[2:3, :] += jnp.sum(y2, axis=(0, 1, 2)).reshape(1, C)
        st_ref[3:4, :] += jnp.sum(y2 * y2, axis=(0, 1, 2)).reshape(1, C)

    @pl.when(p == 2)
    def _phase2():
        sc, sh = coeffs(2, 3, g2_ref, b2_ref)
        a2 = jnp.maximum(y2_ref[j] * sc + sh, 0.0)
        o_ref[...] = jnp.transpose(a2.reshape(Nc, HW, C), (0, 2, 1))


@jax.jit
def kernel(x, conv1_w, bn1_g, bn1_b, conv2_w, bn2_g, bn2_b):
    eps = 1e-5
    N, C, H, W = x.shape
    Hp, Wp = H // 2, W // 2
    Nc = N // _NCHUNK
    Cout = conv1_w.shape[3]
    xb = x.reshape(N, C, Hp, 2 * W)
    lane = jnp.arange(2 * Wp)
    sel = ((lane[:, None] == 2 * (lane[None, :] % Wp) + lane[None, :] // Wp)
           .astype(jnp.bfloat16))
    w1c = conv1_w.transpose(1, 2, 0, 3).reshape(3 * C, 3 * Cout)
    w2c = conv2_w.transpose(1, 2, 0, 3).reshape(3 * Cout, 3 * Cout)
    body = functools.partial(_fused_body, eps=eps)
    last = _NCHUNK - 1
    out = pl.pallas_call(
        body,
        out_shape=jax.ShapeDtypeStruct((N, Cout, Hp * Wp), jnp.float32),
        grid=(3, _NCHUNK),
        in_specs=[
            pl.BlockSpec((Nc, C, Hp, 2 * W),
                         lambda p, j: (jnp.where(p == 0, j, last), 0, 0, 0)),
            pl.BlockSpec((2 * Wp, 2 * Wp), lambda p, j: (0, 0)),
            pl.BlockSpec((3 * C, 3 * Cout), lambda p, j: (0, 0)),
            pl.BlockSpec((3 * Cout, 3 * Cout), lambda p, j: (0, 0)),
            pl.BlockSpec((1, Cout), lambda p, j: (0, 0)),
            pl.BlockSpec((1, Cout), lambda p, j: (0, 0)),
            pl.BlockSpec((1, Cout), lambda p, j: (0, 0)),
            pl.BlockSpec((1, Cout), lambda p, j: (0, 0)),
        ],
        out_specs=pl.BlockSpec((Nc, Cout, Hp * Wp),
                               lambda p, j: (jnp.where(p == 2, j, 0), 0, 0)),
        scratch_shapes=[
            pltpu.VMEM((_NCHUNK, Nc, Hp, Wp, Cout), jnp.float32),
            pltpu.VMEM((_NCHUNK, Nc, Hp, Wp, Cout), jnp.float32),
            pltpu.VMEM((4, Cout), jnp.float32),
        ],
        compiler_params=pltpu.CompilerParams(
            dimension_semantics=("arbitrary", "arbitrary")),
    )(xb, sel, w1c.astype(jnp.bfloat16), w2c.astype(jnp.bfloat16),
      bn1_g.reshape(1, Cout), bn1_b.reshape(1, Cout),
      bn2_g.reshape(1, Cout), bn2_b.reshape(1, Cout))
    return out.reshape(N, Cout, Hp, Wp)
